# baseline (speedup 1.0000x reference)
import functools

import jax
import jax.numpy as jnp
from jax import lax
from jax.experimental import pallas as pl
from jax.experimental.pallas import tpu as pltpu
from jax.experimental.pallas import tpu_sc as plsc

E = 800000
D = 64
NROWS = 60
NC = 2
NS = 16
NW = NC * NS
C = 640
NCHUNKS = E // C
MAX_ITERS = (NCHUNKS + NW - 1) // NW


def _lut_body(w0_ref, w1_ref, w2_ref, lut_ref):
    r = lax.broadcasted_iota(jnp.int32, (NROWS, 1), 0)
    a0 = (r // 12 == lax.broadcasted_iota(jnp.int32, (NROWS, 5), 1)).astype(jnp.float32)
    a1 = ((r % 12) // 2 == lax.broadcasted_iota(jnp.int32, (NROWS, 6), 1)).astype(jnp.float32)
    a2 = (r % 2 == lax.broadcasted_iota(jnp.int32, (NROWS, 2), 1)).astype(jnp.float32)
    f32 = jnp.float32
    lut_ref[...] = (
        jnp.dot(a0, w0_ref[...], preferred_element_type=f32)
        + jnp.dot(a1, w1_ref[...], preferred_element_type=f32)
        + jnp.dot(a2, w2_ref[...], preferred_element_type=f32)
    )


_build_lut = pl.pallas_call(
    _lut_body,
    out_shape=jax.ShapeDtypeStruct((NROWS, D), jnp.float32),
)


@functools.cache
def _make_sc_lookup():
    @functools.partial(
        pl.kernel,
        out_type=jax.ShapeDtypeStruct((E, D), jnp.float32),
        mesh=plsc.VectorSubcoreMesh(
            core_axis_name="c", subcore_axis_name="s",
            num_cores=NC, num_subcores=NS,
        ),
        scratch_types=[
            pltpu.VMEM((NROWS, D), jnp.float32),
            pltpu.VMEM((C, 3), jnp.int32),
            pltpu.VMEM((C, D), jnp.float32),
        ],
        compiler_params=pltpu.CompilerParams(
            use_tc_tiling_on_sc=False, needs_layout_passes=False
        ),
    )
    def _sc_lookup(x_hbm, lut_hbm, out_hbm, lut_v, x_v, rows_v):
        w = lax.axis_index("s") * NC + lax.axis_index("c")
        pltpu.sync_copy(lut_hbm, lut_v)
        lane_iota = lax.iota(jnp.int32, 16)

        def chunk_body(i, carry):
            cid = w + NW * i

            @pl.when(cid < NCHUNKS)
            def _():
                base = pl.multiple_of(cid * C, 8)
                pltpu.sync_copy(x_hbm.at[pl.ds(base, C)], x_v)

                @plsc.parallel_loop(0, C // 16, unroll=2)
                def edge_body(v):
                    erow = lane_iota + v * 16
                    g0 = plsc.load_gather(x_v, [erow, jnp.zeros(16, jnp.int32)])
                    g1 = plsc.load_gather(x_v, [erow, jnp.ones(16, jnp.int32)])
                    g2 = plsc.load_gather(x_v, [erow, jnp.full(16, 2, jnp.int32)])
                    rvec = (g0 * 6 + g1) * 2 + g2
                    for lane in range(16):
                        r = rvec[lane]
                        e = v * 16 + lane
                        for g in range(4):
                            sl = pl.ds(g * 16, 16)
                            rows_v[e, sl] = lut_v[r, sl]

                pltpu.sync_copy(rows_v, out_hbm.at[pl.ds(base, C)])

            return carry

        lax.fori_loop(0, MAX_ITERS, chunk_body, 0)

    return _sc_lookup


def kernel(x, W0, W1, W2):
    x = x.astype(jnp.int32)
    lut = _build_lut(W0, W1, W2)
    return _make_sc_lookup()(x, lut)


# speedup vs baseline: 1.0744x; 1.0744x over previous
import functools

import jax
import jax.numpy as jnp
from jax import lax
from jax.experimental import pallas as pl
from jax.experimental.pallas import tpu as pltpu
from jax.experimental.pallas import tpu_sc as plsc

E = 800000
D = 64
NROWS = 60
NC = 2
NS = 16
NW = NC * NS
C = 640
NCHUNKS = E // C
MAX_ITERS = (NCHUNKS + NW - 1) // NW


def _lut_body(w0_ref, w1_ref, w2_ref, lut_ref):
    r = lax.broadcasted_iota(jnp.int32, (NROWS, 1), 0)
    a0 = (r // 12 == lax.broadcasted_iota(jnp.int32, (NROWS, 5), 1)).astype(jnp.float32)
    a1 = ((r % 12) // 2 == lax.broadcasted_iota(jnp.int32, (NROWS, 6), 1)).astype(jnp.float32)
    a2 = (r % 2 == lax.broadcasted_iota(jnp.int32, (NROWS, 2), 1)).astype(jnp.float32)
    f32 = jnp.float32
    lut_ref[...] = (
        jnp.dot(a0, w0_ref[...], preferred_element_type=f32)
        + jnp.dot(a1, w1_ref[...], preferred_element_type=f32)
        + jnp.dot(a2, w2_ref[...], preferred_element_type=f32)
    )


_build_lut = pl.pallas_call(
    _lut_body,
    out_shape=jax.ShapeDtypeStruct((NROWS, D), jnp.float32),
)


@functools.cache
def _make_sc_lookup():
    @functools.partial(
        pl.kernel,
        out_type=jax.ShapeDtypeStruct((E, D), jnp.float32),
        mesh=plsc.VectorSubcoreMesh(
            core_axis_name="c", subcore_axis_name="s",
            num_cores=NC, num_subcores=NS,
        ),
        scratch_types=[
            pltpu.VMEM((NROWS, D), jnp.float32),
            pltpu.VMEM((C * 3,), jnp.int32),
            pltpu.VMEM((C, D), jnp.float32),
        ],
        compiler_params=pltpu.CompilerParams(
            use_tc_tiling_on_sc=False, needs_layout_passes=False
        ),
    )
    def _sc_lookup(x_hbm, lut_hbm, out_hbm, lut_v, x_v, rows_v):
        w = lax.axis_index("s") * NC + lax.axis_index("c")
        pltpu.sync_copy(lut_hbm, lut_v)
        lane_iota = lax.iota(jnp.int32, 16)

        def chunk_body(i, carry):
            cid = w + NW * i

            @pl.when(cid < NCHUNKS)
            def _():
                base = pl.multiple_of(cid * C, 8)
                pltpu.sync_copy(x_hbm.at[pl.ds(base * 3, C * 3)], x_v)

                @plsc.parallel_loop(0, C // 16, unroll=2)
                def edge_body(v):
                    pos = lane_iota * 3 + v * 48
                    g0 = plsc.load_gather(x_v, [pos])
                    g1 = plsc.load_gather(x_v, [pos + 1])
                    g2 = plsc.load_gather(x_v, [pos + 2])
                    rvec = (g0 * 6 + g1) * 2 + g2
                    for lane in range(16):
                        r = rvec[lane]
                        e = v * 16 + lane
                        for g in range(4):
                            sl = pl.ds(g * 16, 16)
                            rows_v[e, sl] = lut_v[r, sl]

                pltpu.sync_copy(rows_v, out_hbm.at[pl.ds(base, C)])

            return carry

        lax.fori_loop(0, MAX_ITERS, chunk_body, 0)

    return _sc_lookup


def kernel(x, W0, W1, W2):
    x = x.astype(jnp.int32).reshape(-1)
    lut = _build_lut(W0, W1, W2)
    return _make_sc_lookup()(x, lut)


# speedup vs baseline: 5.8375x; 5.4331x over previous
import functools

import jax
import jax.numpy as jnp
from jax import lax
from jax.experimental import pallas as pl
from jax.experimental.pallas import tpu as pltpu
from jax.experimental.pallas import tpu_sc as plsc

E = 800000
D = 64
NROWS = 60
NC = 2
NS = 16
NW = NC * NS
C = 640
NCHUNKS = E // C
MAX_ITERS = (NCHUNKS + NW - 1) // NW


def _lut_body(w0_ref, w1_ref, w2_ref, lut_ref):
    r = lax.broadcasted_iota(jnp.int32, (NROWS, 1), 0)
    a0 = (r // 12 == lax.broadcasted_iota(jnp.int32, (NROWS, 5), 1)).astype(jnp.float32)
    a1 = ((r % 12) // 2 == lax.broadcasted_iota(jnp.int32, (NROWS, 6), 1)).astype(jnp.float32)
    a2 = (r % 2 == lax.broadcasted_iota(jnp.int32, (NROWS, 2), 1)).astype(jnp.float32)
    f32 = jnp.float32
    lut_ref[...] = (
        jnp.dot(a0, w0_ref[...], preferred_element_type=f32)
        + jnp.dot(a1, w1_ref[...], preferred_element_type=f32)
        + jnp.dot(a2, w2_ref[...], preferred_element_type=f32)
    )


_build_lut = pl.pallas_call(
    _lut_body,
    out_shape=jax.ShapeDtypeStruct((NROWS, D), jnp.float32),
)


@functools.cache
def _make_sc_lookup():
    @functools.partial(
        pl.kernel,
        out_type=jax.ShapeDtypeStruct((E, D), jnp.float32),
        mesh=plsc.VectorSubcoreMesh(
            core_axis_name="c", subcore_axis_name="s",
            num_cores=NC, num_subcores=NS,
        ),
        scratch_types=[
            pltpu.VMEM((NROWS, D), jnp.float32),
            pltpu.VMEM((C,), jnp.int32),
            pltpu.VMEM((C,), jnp.int32),
            pltpu.VMEM((C,), jnp.int32),
            pltpu.VMEM((C, D), jnp.float32),
        ],
        compiler_params=pltpu.CompilerParams(use_tc_tiling_on_sc=True),
    )
    def _sc_lookup(x0_hbm, x1_hbm, x2_hbm, lut_hbm, out_hbm,
                   lut_v, x0_v, x1_v, x2_v, rows_v):
        w = lax.axis_index("s") * NC + lax.axis_index("c")
        pltpu.sync_copy(lut_hbm, lut_v)

        def chunk_body(i, carry):
            cid = w + NW * i

            @pl.when(cid < NCHUNKS)
            def _():
                base = pl.multiple_of(cid * C, 128)
                pltpu.sync_copy(x0_hbm.at[pl.ds(base, C)], x0_v)
                pltpu.sync_copy(x1_hbm.at[pl.ds(base, C)], x1_v)
                pltpu.sync_copy(x2_hbm.at[pl.ds(base, C)], x2_v)

                @plsc.parallel_loop(0, C // 16, unroll=2)
                def edge_body(v):
                    sl16 = pl.ds(v * 16, 16)
                    rvec = (x0_v[sl16] * 6 + x1_v[sl16]) * 2 + x2_v[sl16]
                    for lane in range(16):
                        r = rvec[lane]
                        e = v * 16 + lane
                        for g in range(4):
                            sl = pl.ds(g * 16, 16)
                            rows_v[e, sl] = lut_v[r, sl]

                pltpu.sync_copy(rows_v, out_hbm.at[pl.ds(base, C)])

            return carry

        lax.fori_loop(0, MAX_ITERS, chunk_body, 0)

    return _sc_lookup


def kernel(x, W0, W1, W2):
    x = x.astype(jnp.int32)
    lut = _build_lut(W0, W1, W2)
    return _make_sc_lookup()(x[:, 0], x[:, 1], x[:, 2], lut)
